# baseline (speedup 1.0000x reference)
import functools

import jax
import jax.numpy as jnp
from jax import lax
from jax.experimental import pallas as pl
from jax.experimental.pallas import tpu as pltpu, tpu_sc as plsc

_NUM_CORES = 2
_NUM_SUBCORES = 16
_NUM_WORKERS = _NUM_CORES * _NUM_SUBCORES


@functools.cache
def _sc_copy(n):
    chunk = n // _NUM_WORKERS
    mesh = plsc.VectorSubcoreMesh(core_axis_name="c", subcore_axis_name="s")

    @functools.partial(
        pl.kernel,
        mesh=mesh,
        out_type=(
            jax.ShapeDtypeStruct((n,), jnp.float32),
            jax.ShapeDtypeStruct((n,), jnp.float32),
        ),
        scratch_types=[
            pltpu.VMEM((chunk,), jnp.float32),
            pltpu.VMEM((chunk,), jnp.float32),
        ],
    )
    def body(kx_hbm, vx_hbm, k_out_hbm, v_out_hbm, kbuf, vbuf):
        wid = lax.axis_index("s") * _NUM_CORES + lax.axis_index("c")
        base = wid * chunk
        pltpu.sync_copy(kx_hbm.at[pl.ds(base, chunk)], kbuf)
        pltpu.sync_copy(vx_hbm.at[pl.ds(base, chunk)], vbuf)
        pltpu.sync_copy(kbuf, k_out_hbm.at[pl.ds(base, chunk)])
        pltpu.sync_copy(vbuf, v_out_hbm.at[pl.ds(base, chunk)])

    return body


def kernel(kx, vx, k_cache, v_cache):
    del k_cache, v_cache
    shape = kx.shape
    n = kx.size
    k_flat, v_flat = _sc_copy(n)(kx.reshape(n), vx.reshape(n))
    return k_flat.reshape(shape), v_flat.reshape(shape)
